# baseline (speedup 1.0000x reference)
import functools

import jax
import jax.numpy as jnp
from jax import lax
from jax.experimental import pallas as pl
from jax.experimental.pallas import tpu as pltpu
from jax.experimental.pallas import tpu_sc as plsc

H = 128
LANES = 16
NC = 2
NS = 16
NW = NC * NS


def _sc_embed(idx_flat, word_table, pos_table):
    n_rows = idx_flat.shape[0]
    lseq = pos_table.shape[0]
    rows_per_w = n_rows // NW
    nb = rows_per_w // lseq
    nj = nb // 2
    chunk_bytes_split = (128, lseq - 128)

    mesh = plsc.VectorSubcoreMesh(
        core_axis_name="c", subcore_axis_name="s",
        num_cores=NC, num_subcores=NS)

    @functools.partial(
        pl.kernel,
        out_type=jax.ShapeDtypeStruct((n_rows, H), jnp.float32),
        mesh=mesh,
        scratch_types=[
            pltpu.VMEM((lseq, H), jnp.float32),
            pltpu.VMEM((rows_per_w,), jnp.int32),
            pltpu.VMEM((lseq, H), jnp.float32),
            pltpu.VMEM((lseq, H), jnp.float32),
            pltpu.VMEM((lseq, H), jnp.float32),
            pltpu.SemaphoreType.DMA,
            pltpu.SemaphoreType.DMA,
            pltpu.SemaphoreType.DMA,
            pltpu.SemaphoreType.DMA,
            pltpu.SemaphoreType.DMA,
            pltpu.SemaphoreType.DMA,
        ],
    )
    def k(idx_hbm, word_hbm, pos_hbm, out_hbm,
          pos_v, idx_v, wb0, wb1, wb2, gs0, gs1, gs2, ws0, ws1, ws2):
        wid = lax.axis_index("s") * NC + lax.axis_index("c")
        base = wid * rows_per_w
        pltpu.sync_copy(pos_hbm, pos_v)
        pltpu.sync_copy(idx_hbm.at[pl.ds(base, rows_per_w)], idx_v)

        wbufs = (wb0, wb1, wb2)
        gsems = (gs0, gs1, gs2)
        wsems = (ws0, ws1, ws2)

        def start_gather(g, b):
            off = pl.multiple_of(g * lseq, 8)
            n0, n1 = chunk_bytes_split
            c0 = pltpu.async_copy(
                word_hbm.at[idx_v.at[pl.ds(off, n0)]],
                wbufs[b].at[pl.ds(0, n0)], gsems[b])
            c1 = pltpu.async_copy(
                word_hbm.at[idx_v.at[pl.ds(off + n0, n1)]],
                wbufs[b].at[pl.ds(n0, n1)], gsems[b])
            return c0, c1

        def drain_gather(b):
            n0, n1 = chunk_bytes_split
            pltpu.make_async_copy(
                word_hbm.at[idx_v.at[pl.ds(0, n0)]],
                wbufs[b].at[pl.ds(0, n0)], gsems[b]).wait()
            pltpu.make_async_copy(
                word_hbm.at[idx_v.at[pl.ds(n0, n1)]],
                wbufs[b].at[pl.ds(n0, n1)], gsems[b]).wait()

        def compute(b):
            buf = wbufs[b]

            @plsc.parallel_loop(0, lseq, 1, unroll=4)
            def _(r):
                for h0 in range(0, H, LANES):
                    sl = pl.ds(h0, LANES)
                    buf[r, sl] = jnp.maximum(buf[r, sl] + pos_v[r, sl], 0.0)

        def start_write(g, b):
            off = pl.multiple_of(base + g * lseq, 8)
            return pltpu.async_copy(
                wbufs[b], out_hbm.at[pl.ds(off, lseq)], wsems[b])

        def drain_write(b):
            pltpu.make_async_copy(
                wbufs[b], out_hbm.at[pl.ds(base, lseq)], wsems[b]).wait()

        def steady(i, t):
            tn = (t + 2) % 3
            drain_gather(t)
            compute(t)
            drain_write(tn)
            start_write(i, t)
            start_gather(i + 2, tn)

        def finish(i, t):
            drain_gather(t)
            compute(t)
            start_write(i, t)

        start_gather(0, 0)
        start_gather(1, 1)
        start_gather(2, 2)
        drain_gather(0)
        compute(0)
        start_write(0, 0)
        nj3 = (nb - 4) // 3

        def body(j, _):
            a = j * 3
            steady(a + 1, 1)
            steady(a + 2, 2)
            steady(a + 3, 0)
            return 0

        lax.fori_loop(0, nj3, body, 0)
        steady(nb - 4, (nb - 4) % 3)
        steady(nb - 3, (nb - 3) % 3)
        finish(nb - 2, (nb - 2) % 3)
        finish(nb - 1, (nb - 1) % 3)
        drain_write((nb - 3) % 3)
        drain_write((nb - 2) % 3)
        drain_write((nb - 1) % 3)

    return k(idx_flat, word_table, pos_table)


def kernel(input_seq, word_table, pos_table):
    B, L = input_seq.shape
    idx_flat = input_seq.reshape(-1).astype(jnp.int32)
    out = _sc_embed(idx_flat, word_table, pos_table)
    return out.reshape(B, L, word_table.shape[1])


# speedup vs baseline: 1.0074x; 1.0074x over previous
import functools

import jax
import jax.numpy as jnp
from jax import lax
from jax.experimental import pallas as pl
from jax.experimental.pallas import tpu as pltpu
from jax.experimental.pallas import tpu_sc as plsc

H = 128
LANES = 16
NC = 2
NS = 16
NW = NC * NS
NBUF = 4


def _sc_embed(idx_flat, word_table, pos_table):
    n_rows = idx_flat.shape[0]
    lseq = pos_table.shape[0]
    rows_per_w = n_rows // NW
    nb = rows_per_w // lseq
    n0, n1 = 128, lseq - 128

    mesh = plsc.VectorSubcoreMesh(
        core_axis_name="c", subcore_axis_name="s",
        num_cores=NC, num_subcores=NS)

    @functools.partial(
        pl.kernel,
        out_type=jax.ShapeDtypeStruct((n_rows, H), jnp.float32),
        mesh=mesh,
        scratch_types=[
            pltpu.VMEM((lseq, H), jnp.float32),
            [pltpu.VMEM((lseq, H), jnp.float32) for _ in range(NBUF)],
            [pltpu.VMEM((lseq,), jnp.int32) for _ in range(NBUF)],
            [pltpu.SemaphoreType.DMA for _ in range(NBUF)],
            [pltpu.SemaphoreType.DMA for _ in range(NBUF)],
            [pltpu.SemaphoreType.DMA for _ in range(NBUF)],
        ],
    )
    def k(idx_hbm, word_hbm, pos_hbm, out_hbm,
          pos_v, wbufs, ibufs, gsems, wsems, isems):
        wid = lax.axis_index("s") * NC + lax.axis_index("c")
        base = wid * rows_per_w
        pltpu.sync_copy(pos_hbm, pos_v)

        def start_idx(g, b):
            off = pl.multiple_of(base + g * lseq, 8)
            pltpu.async_copy(idx_hbm.at[pl.ds(off, lseq)], ibufs[b], isems[b])

        def drain_idx(b):
            pltpu.make_async_copy(
                idx_hbm.at[pl.ds(base, lseq)], ibufs[b], isems[b]).wait()

        def start_gather(b):
            pltpu.async_copy(
                word_hbm.at[ibufs[b].at[pl.ds(0, n0)]],
                wbufs[b].at[pl.ds(0, n0)], gsems[b])
            pltpu.async_copy(
                word_hbm.at[ibufs[b].at[pl.ds(n0, n1)]],
                wbufs[b].at[pl.ds(n0, n1)], gsems[b])

        def drain_gather(b):
            pltpu.make_async_copy(
                word_hbm.at[ibufs[b].at[pl.ds(0, n0)]],
                wbufs[b].at[pl.ds(0, n0)], gsems[b]).wait()
            pltpu.make_async_copy(
                word_hbm.at[ibufs[b].at[pl.ds(n0, n1)]],
                wbufs[b].at[pl.ds(n0, n1)], gsems[b]).wait()

        def compute(b):
            buf = wbufs[b]

            @plsc.parallel_loop(0, lseq, 1, unroll=4)
            def _(r):
                for h0 in range(0, H, LANES):
                    sl = pl.ds(h0, LANES)
                    buf[r, sl] = jnp.maximum(buf[r, sl] + pos_v[r, sl], 0.0)

        def start_write(g, b):
            off = pl.multiple_of(base + g * lseq, 8)
            pltpu.async_copy(wbufs[b], out_hbm.at[pl.ds(off, lseq)], wsems[b])

        def drain_write(b):
            pltpu.make_async_copy(
                wbufs[b], out_hbm.at[pl.ds(base, lseq)], wsems[b]).wait()

        def steady(i, t, prefetch_gather, prefetch_idx):
            tg = (t + 2) % NBUF
            drain_gather(t)
            if prefetch_idx:
                start_idx(i + 4, t)
            if prefetch_gather:
                drain_write(tg)
                drain_idx(tg)
                start_gather(tg)
            compute(t)
            start_write(i, t)

        for g in range(NBUF):
            start_idx(g, g)
        drain_idx(0)
        start_gather(0)
        drain_idx(1)
        start_gather(1)
        drain_gather(0)
        drain_idx(2)
        start_gather(2)
        start_idx(NBUF, 0)
        compute(0)
        start_write(0, 0)
        drain_gather(1)
        drain_idx(3)
        start_gather(3)
        start_idx(NBUF + 1, 1)
        compute(1)
        start_write(1, 1)
        nj4 = (nb - 8) // NBUF

        def body(j, _):
            a = j * NBUF + 2
            steady(a, 2, True, True)
            steady(a + 1, 3, True, True)
            steady(a + 2, 0, True, True)
            steady(a + 3, 1, True, True)
            return 0

        lax.fori_loop(0, nj4, body, 0)
        steady(nb - 6, (nb - 6) % NBUF, True, True)
        steady(nb - 5, (nb - 5) % NBUF, True, True)
        steady(nb - 4, (nb - 4) % NBUF, True, False)
        steady(nb - 3, (nb - 3) % NBUF, True, False)
        steady(nb - 2, (nb - 2) % NBUF, False, False)
        steady(nb - 1, (nb - 1) % NBUF, False, False)
        for g in range(nb - NBUF, nb):
            drain_write(g % NBUF)

    return k(idx_flat, word_table, pos_table)


def kernel(input_seq, word_table, pos_table):
    B, L = input_seq.shape
    idx_flat = input_seq.reshape(-1).astype(jnp.int32)
    out = _sc_embed(idx_flat, word_table, pos_table)
    return out.reshape(B, L, word_table.shape[1])


# speedup vs baseline: 1.0218x; 1.0143x over previous
import functools

import jax
import jax.numpy as jnp
from jax import lax
from jax.experimental import pallas as pl
from jax.experimental.pallas import tpu as pltpu
from jax.experimental.pallas import tpu_sc as plsc

H = 128
LANES = 16
NC = 2
NS = 16
NW = NC * NS
NBUF = 4


def _sc_embed(idx_flat, word_table, pos_table):
    n_rows = idx_flat.shape[0]
    lseq = pos_table.shape[0]
    rows_per_w = n_rows // NW
    nb = rows_per_w // lseq
    n0, n1 = 128, lseq - 128

    mesh = plsc.VectorSubcoreMesh(
        core_axis_name="c", subcore_axis_name="s",
        num_cores=NC, num_subcores=NS)

    @functools.partial(
        pl.kernel,
        out_type=jax.ShapeDtypeStruct((n_rows, H), jnp.float32),
        mesh=mesh,
        scratch_types=[
            pltpu.VMEM((lseq, H), jnp.float32),
            [pltpu.VMEM((lseq, H), jnp.float32) for _ in range(NBUF)],
            [pltpu.VMEM((lseq,), jnp.int32) for _ in range(NBUF)],
            [pltpu.SemaphoreType.DMA for _ in range(NBUF)],
            [pltpu.SemaphoreType.DMA for _ in range(NBUF)],
            [pltpu.SemaphoreType.DMA for _ in range(NBUF)],
            pltpu.SemaphoreType.DMA,
        ],
    )
    def k(idx_hbm, word_hbm, pos_hbm, out_hbm,
          pos_v, wbufs, ibufs, gsems, wsems, isems, psem):
        wid = lax.axis_index("s") * NC + lax.axis_index("c")
        base = wid * rows_per_w
        pos_copy = pltpu.async_copy(pos_hbm, pos_v, psem)

        def start_idx(g, b):
            off = pl.multiple_of(base + g * lseq, 8)
            pltpu.async_copy(idx_hbm.at[pl.ds(off, lseq)], ibufs[b], isems[b])

        def drain_idx(b):
            pltpu.make_async_copy(
                idx_hbm.at[pl.ds(base, lseq)], ibufs[b], isems[b]).wait()

        def start_gather(b):
            pltpu.async_copy(
                word_hbm.at[ibufs[b].at[pl.ds(0, n0)]],
                wbufs[b].at[pl.ds(0, n0)], gsems[b])
            pltpu.async_copy(
                word_hbm.at[ibufs[b].at[pl.ds(n0, n1)]],
                wbufs[b].at[pl.ds(n0, n1)], gsems[b])

        def drain_gather(b):
            pltpu.make_async_copy(
                word_hbm.at[ibufs[b].at[pl.ds(0, n0)]],
                wbufs[b].at[pl.ds(0, n0)], gsems[b]).wait()
            pltpu.make_async_copy(
                word_hbm.at[ibufs[b].at[pl.ds(n0, n1)]],
                wbufs[b].at[pl.ds(n0, n1)], gsems[b]).wait()

        def compute(b):
            buf = wbufs[b]

            @plsc.parallel_loop(0, lseq, 1, unroll=4)
            def _(r):
                for h0 in range(0, H, LANES):
                    sl = pl.ds(h0, LANES)
                    buf[r, sl] = jnp.maximum(buf[r, sl] + pos_v[r, sl], 0.0)

        def start_write(g, b):
            off = pl.multiple_of(base + g * lseq, 8)
            pltpu.async_copy(wbufs[b], out_hbm.at[pl.ds(off, lseq)], wsems[b])

        def drain_write(b):
            pltpu.make_async_copy(
                wbufs[b], out_hbm.at[pl.ds(base, lseq)], wsems[b]).wait()

        def steady(i, t, prefetch_gather, prefetch_idx):
            tg = (t + 2) % NBUF
            drain_gather(t)
            if prefetch_idx:
                start_idx(i + 4, t)
            if prefetch_gather:
                drain_write(tg)
                drain_idx(tg)
                start_gather(tg)
            compute(t)
            start_write(i, t)

        for g in range(NBUF):
            start_idx(g, g)
        drain_idx(0)
        start_gather(0)
        drain_idx(1)
        start_gather(1)
        pos_copy.wait()
        drain_gather(0)
        drain_idx(2)
        start_gather(2)
        start_idx(NBUF, 0)
        compute(0)
        start_write(0, 0)
        drain_gather(1)
        drain_idx(3)
        start_gather(3)
        start_idx(NBUF + 1, 1)
        compute(1)
        start_write(1, 1)
        nj4 = (nb - 8) // NBUF

        def body(j, _):
            a = j * NBUF + 2
            steady(a, 2, True, True)
            steady(a + 1, 3, True, True)
            steady(a + 2, 0, True, True)
            steady(a + 3, 1, True, True)
            return 0

        lax.fori_loop(0, nj4, body, 0)
        steady(nb - 6, (nb - 6) % NBUF, True, True)
        steady(nb - 5, (nb - 5) % NBUF, True, True)
        steady(nb - 4, (nb - 4) % NBUF, True, False)
        steady(nb - 3, (nb - 3) % NBUF, True, False)
        steady(nb - 2, (nb - 2) % NBUF, False, False)
        steady(nb - 1, (nb - 1) % NBUF, False, False)
        for g in range(nb - NBUF, nb):
            drain_write(g % NBUF)

    return k(idx_flat, word_table, pos_table)


def kernel(input_seq, word_table, pos_table):
    B, L = input_seq.shape
    idx_flat = input_seq.reshape(-1).astype(jnp.int32)
    out = _sc_embed(idx_flat, word_table, pos_table)
    return out.reshape(B, L, word_table.shape[1])
